# Initial kernel scaffold; baseline (speedup 1.0000x reference)
import jax
import jax.numpy as jnp
from jax.experimental import pallas as pl


def kernel(log_seqs, user_ids, item_table, user_table):
    raise NotImplementedError("write your pallas kernel here")


# speedup vs baseline: 1.2326x; 1.2326x over previous
import functools

import jax
import jax.numpy as jnp
from jax import lax
from jax.experimental import pallas as pl
from jax.experimental.pallas import tpu as pltpu
from jax.experimental.pallas import tpu_sc as plsc

_ITEM_NUM = 1000000
_USER_NUM = 100000
_IH = 48
_UH = 16
_SSE_PROB = 0.08
_B = 4096
_L = 200

_NC = 2
_NS = 16
_NW = _NC * _NS
_B_PER_W = _B // _NW
_CB = 2
_R = _CB * _L
_NCHUNK = _B_PER_W // _CB


def _sse_uids(user_ids):
    key = jax.random.key(42)
    ku, kr = jax.random.split(key)
    probs = jax.random.uniform(ku, user_ids.shape)
    rand_ids = jax.random.randint(kr, user_ids.shape, 1, _USER_NUM + 1)
    rand_ids = rand_ids.astype(user_ids.dtype)
    return jnp.where(probs < _SSE_PROB, rand_ids, user_ids)


def _body(seqs_hbm, uids_hbm, item_hbm, user_hbm, out_hbm,
          uidx_v, urows_v, idx_v, rows_v, ubuf_v, sem):
    wid = lax.axis_index("s") * _NC + lax.axis_index("c")
    base_b = wid * _B_PER_W

    pltpu.sync_copy(uids_hbm.at[pl.ds(base_b, _B_PER_W)], uidx_v)
    pltpu.async_copy(user_hbm.at[uidx_v], urows_v, sem).wait()

    def chunk_body(ci, carry):
        b0 = base_b + ci * _CB
        row0 = b0 * _L
        pltpu.sync_copy(seqs_hbm.at[pl.ds(row0, _R)], idx_v)
        item_cp = pltpu.async_copy(item_hbm.at[idx_v], rows_v, sem)

        def fill_b(s, c2):
            urow = urows_v[ci * _CB + s, :]

            def fill_row(j, c3):
                ubuf_v[s * _L + j, :] = urow
                return c3

            return lax.fori_loop(0, _L, fill_row, c2)

        lax.fori_loop(0, _CB, fill_b, 0)

        item_cp.wait()
        pltpu.sync_copy(rows_v, out_hbm.at[pl.ds(row0, _R), pl.ds(0, _IH)])
        pltpu.sync_copy(ubuf_v, out_hbm.at[pl.ds(row0, _R), pl.ds(_IH, _UH)])
        return carry

    lax.fori_loop(0, _NCHUNK, chunk_body, 0)


@jax.jit
def _sc_embed(seqs1d, uids, item_table, user_table):
    mesh = plsc.VectorSubcoreMesh(core_axis_name="c", subcore_axis_name="s")
    f = pl.kernel(
        _body,
        out_type=jax.ShapeDtypeStruct((_B * _L, _IH + _UH), jnp.float32),
        mesh=mesh,
        scratch_types=[
            pltpu.VMEM((_B_PER_W,), jnp.int32),
            pltpu.VMEM((_B_PER_W, _UH), jnp.float32),
            pltpu.VMEM((_R,), jnp.int32),
            pltpu.VMEM((_R, _IH), jnp.float32),
            pltpu.VMEM((_R, _UH), jnp.float32),
            pltpu.SemaphoreType.DMA,
        ],
        compiler_params=pltpu.CompilerParams(use_tc_tiling_on_sc=False),
    )
    return f(seqs1d, uids, item_table, user_table)


def kernel(log_seqs, user_ids, item_table, user_table):
    uids = _sse_uids(user_ids).astype(jnp.int32)
    seqs1d = log_seqs.reshape(-1).astype(jnp.int32)
    out2d = _sc_embed(seqs1d, uids, item_table, user_table)
    return out2d.reshape(_B, _L, _IH + _UH)


# speedup vs baseline: 1.2379x; 1.0043x over previous
import jax
import jax.numpy as jnp
from jax import lax
from jax.experimental import pallas as pl
from jax.experimental.pallas import tpu as pltpu
from jax.experimental.pallas import tpu_sc as plsc

_ITEM_NUM = 1000000
_USER_NUM = 100000
_IH = 48
_UH = 16
_SSE_PROB = 0.08
_B = 4096
_L = 200

_NC = 2
_NS = 16
_NW = _NC * _NS
_B_PER_W = _B // _NW
_CB = 2
_R = _CB * _L
_NCHUNK = _B_PER_W // _CB
_ROWS_PER_W = _B_PER_W * _L
_NBUF = 3


def _sse_uids(user_ids):
    key = jax.random.key(42)
    ku, kr = jax.random.split(key)
    probs = jax.random.uniform(ku, user_ids.shape)
    rand_ids = jax.random.randint(kr, user_ids.shape, 1, _USER_NUM + 1)
    rand_ids = rand_ids.astype(user_ids.dtype)
    return jnp.where(probs < _SSE_PROB, rand_ids, user_ids)


def _body(seqs_hbm, urow_idx_hbm, item_hbm, user_hbm, out_hbm,
          idxi_v, idxu_v, rows_v, urows_v, gsems, wsems):
    wid = lax.axis_index("s") * _NC + lax.axis_index("c")
    base_row = wid * _ROWS_PER_W

    pltpu.sync_copy(seqs_hbm.at[pl.ds(base_row, _ROWS_PER_W)], idxi_v)
    pltpu.sync_copy(urow_idx_hbm.at[pl.ds(base_row, _ROWS_PER_W)], idxu_v)

    def fire(ci, k):
        off = ci * _R
        pltpu.async_copy(
            item_hbm.at[idxi_v.at[pl.ds(off, _R)]], rows_v.at[k],
            gsems.at[k])
        pltpu.async_copy(
            user_hbm.at[idxu_v.at[pl.ds(off, _R)]], urows_v.at[k],
            gsems.at[k])

    def wait_gathers(ci, k):
        off = ci * _R
        pltpu.make_async_copy(
            item_hbm.at[idxi_v.at[pl.ds(off, _R)]], rows_v.at[k],
            gsems.at[k]).wait()
        pltpu.make_async_copy(
            user_hbm.at[idxu_v.at[pl.ds(off, _R)]], urows_v.at[k],
            gsems.at[k]).wait()

    def out_slices(ci):
        row0 = base_row + ci * _R
        return (out_hbm.at[pl.ds(row0, _R), pl.ds(0, _IH)],
                out_hbm.at[pl.ds(row0, _R), pl.ds(_IH, _UH)])

    for k in range(_NBUF):
        fire(k, k)

    def group_body(g, carry):
        for k in range(_NBUF):
            ci = g * _NBUF + k
            wait_gathers(ci, k)
            oi, ou = out_slices(ci)
            pltpu.async_copy(rows_v.at[k], oi, wsems.at[k])
            pltpu.async_copy(urows_v.at[k], ou, wsems.at[k])
            pltpu.make_async_copy(rows_v.at[k], oi, wsems.at[k]).wait()
            pltpu.make_async_copy(urows_v.at[k], ou, wsems.at[k]).wait()

            @pl.when(ci + _NBUF < _NCHUNK)
            def _():
                fire(ci + _NBUF, k)
        return carry

    lax.fori_loop(0, _NCHUNK // _NBUF, group_body, 0)

    for ci in range((_NCHUNK // _NBUF) * _NBUF, _NCHUNK):
        k = ci % _NBUF
        wait_gathers(ci, k)
        oi, ou = out_slices(ci)
        pltpu.async_copy(rows_v.at[k], oi, wsems.at[k])
        pltpu.async_copy(urows_v.at[k], ou, wsems.at[k])
        pltpu.make_async_copy(rows_v.at[k], oi, wsems.at[k]).wait()
        pltpu.make_async_copy(urows_v.at[k], ou, wsems.at[k]).wait()


@jax.jit
def _sc_embed(seqs1d, urow_idx, item_table, user_table):
    mesh = plsc.VectorSubcoreMesh(core_axis_name="c", subcore_axis_name="s")
    f = pl.kernel(
        _body,
        out_type=jax.ShapeDtypeStruct((_B * _L, _IH + _UH), jnp.float32),
        mesh=mesh,
        scratch_types=[
            pltpu.VMEM((_ROWS_PER_W,), jnp.int32),
            pltpu.VMEM((_ROWS_PER_W,), jnp.int32),
            pltpu.VMEM((_NBUF, _R, _IH), jnp.float32),
            pltpu.VMEM((_NBUF, _R, _UH), jnp.float32),
            pltpu.SemaphoreType.DMA((_NBUF,)),
            pltpu.SemaphoreType.DMA((_NBUF,)),
        ],
        compiler_params=pltpu.CompilerParams(use_tc_tiling_on_sc=False),
    )
    return f(seqs1d, urow_idx, item_table, user_table)


def kernel(log_seqs, user_ids, item_table, user_table):
    uids = _sse_uids(user_ids).astype(jnp.int32)
    urow_idx = jnp.broadcast_to(uids[:, None], (_B, _L)).reshape(-1)
    seqs1d = log_seqs.reshape(-1).astype(jnp.int32)
    out2d = _sc_embed(seqs1d, urow_idx, item_table, user_table)
    return out2d.reshape(_B, _L, _IH + _UH)


# speedup vs baseline: 1.2386x; 1.0006x over previous
import jax
import jax.numpy as jnp
from jax import lax
from jax.experimental import pallas as pl
from jax.experimental.pallas import tpu as pltpu
from jax.experimental.pallas import tpu_sc as plsc

_ITEM_NUM = 1000000
_USER_NUM = 100000
_IH = 48
_UH = 16
_SSE_PROB = 0.08
_B = 4096
_L = 200

_NC = 2
_NS = 16
_NW = _NC * _NS
_B_PER_W = _B // _NW
_CB = 2
_R = _CB * _L
_NCHUNK = _B_PER_W // _CB
_ROWS_PER_W = _B_PER_W * _L
_NBUF = 3


def _sse_uids(user_ids):
    key = jax.random.key(42)
    ku, kr = jax.random.split(key)
    probs = jax.random.uniform(ku, user_ids.shape)
    rand_ids = jax.random.randint(kr, user_ids.shape, 1, _USER_NUM + 1)
    rand_ids = rand_ids.astype(user_ids.dtype)
    return jnp.where(probs < _SSE_PROB, rand_ids, user_ids)


def _body(seqs_hbm, urow_idx_hbm, item_hbm, user_hbm, out_hbm,
          idxi_v, idxu_v, rows_v, urows_v, gsems, wsems):
    wid = lax.axis_index("s") * _NC + lax.axis_index("c")
    base_row = wid * _ROWS_PER_W

    pltpu.sync_copy(seqs_hbm.at[pl.ds(base_row, _ROWS_PER_W)], idxi_v)
    pltpu.sync_copy(urow_idx_hbm.at[pl.ds(base_row, _ROWS_PER_W)], idxu_v)

    def fire(ci, k):
        off = ci * _R
        pltpu.async_copy(
            item_hbm.at[idxi_v.at[pl.ds(off, _R)]], rows_v.at[k],
            gsems.at[k])
        pltpu.async_copy(
            user_hbm.at[idxu_v.at[pl.ds(off, _R)]], urows_v.at[k],
            gsems.at[k])

    def wait_gathers(ci, k):
        off = ci * _R
        pltpu.make_async_copy(
            item_hbm.at[idxi_v.at[pl.ds(off, _R)]], rows_v.at[k],
            gsems.at[k]).wait()
        pltpu.make_async_copy(
            user_hbm.at[idxu_v.at[pl.ds(off, _R)]], urows_v.at[k],
            gsems.at[k]).wait()

    def out_slices(ci):
        row0 = base_row + ci * _R
        return (out_hbm.at[pl.ds(row0, _R), pl.ds(0, _IH)],
                out_hbm.at[pl.ds(row0, _R), pl.ds(_IH, _UH)])

    for k in range(_NBUF):
        fire(k, k)

    def group_body(g, carry):
        for k in range(_NBUF):
            ci = g * _NBUF + k
            wait_gathers(ci, k)
            oi, ou = out_slices(ci)
            pltpu.async_copy(rows_v.at[k], oi, wsems.at[k])
            pltpu.async_copy(urows_v.at[k], ou, wsems.at[k])
            pltpu.make_async_copy(rows_v.at[k], oi, wsems.at[k]).wait()
            pltpu.make_async_copy(urows_v.at[k], ou, wsems.at[k]).wait()

            @pl.when(ci + _NBUF < _NCHUNK)
            def _():
                fire(ci + _NBUF, k)
        return carry

    lax.fori_loop(0, _NCHUNK // _NBUF, group_body, 0)

    for ci in range((_NCHUNK // _NBUF) * _NBUF, _NCHUNK):
        k = ci % _NBUF
        wait_gathers(ci, k)
        oi, ou = out_slices(ci)
        pltpu.async_copy(rows_v.at[k], oi, wsems.at[k])
        pltpu.async_copy(urows_v.at[k], ou, wsems.at[k])
        pltpu.make_async_copy(rows_v.at[k], oi, wsems.at[k]).wait()
        pltpu.make_async_copy(urows_v.at[k], ou, wsems.at[k]).wait()


@jax.jit
def _sc_embed(seqs1d, urow_idx, item_table, user_table):
    mesh = plsc.VectorSubcoreMesh(core_axis_name="c", subcore_axis_name="s")
    f = pl.kernel(
        _body,
        out_type=jax.ShapeDtypeStruct((_B * _L, _IH + _UH), jnp.float32),
        mesh=mesh,
        scratch_types=[
            pltpu.VMEM((_ROWS_PER_W,), jnp.int32),
            pltpu.VMEM((_ROWS_PER_W,), jnp.int32),
            pltpu.VMEM((_NBUF, _R, _IH), jnp.float32),
            pltpu.VMEM((_NBUF, _R, _UH), jnp.float32),
            pltpu.SemaphoreType.DMA((_NBUF,)),
            pltpu.SemaphoreType.DMA((_NBUF,)),
        ],
        compiler_params=pltpu.CompilerParams(use_tc_tiling_on_sc=False),
    )
    return f(seqs1d, urow_idx, item_table, user_table)


def kernel(log_seqs, user_ids, item_table, user_table):
    uids = _sse_uids(user_ids).astype(jnp.int32)
    urow_idx = jnp.broadcast_to(uids[:, None], (_B, _L)).reshape(-1)
    seqs1d = log_seqs.reshape(-1).astype(jnp.int32)
    item2d = jax.lax.optimization_barrier(jnp.swapaxes(item_table, 0, 1))
    item2d = jnp.swapaxes(item2d, 0, 1)
    user2d = jax.lax.optimization_barrier(jnp.swapaxes(user_table, 0, 1))
    user2d = jnp.swapaxes(user2d, 0, 1)
    out2d = _sc_embed(seqs1d, urow_idx, item2d, user2d)
    return out2d.reshape(_B, _L, _IH + _UH)
